# baseline (speedup 1.0000x reference)
import functools
import numpy as np
import jax
from jax import lax
import jax.numpy as jnp
from jax.experimental import pallas as pl
from jax.experimental.pallas import tpu as pltpu
from jax.experimental.pallas import tpu_sc as plsc

HID_ = 128
M_, N_, B_ = 28, 28, 32
T_ = M_ + N_ - 1
Q_ = 64
ROWS_ = M_ * B_

_SC_NBUF = 3


def _sc_gather(table, idx, csize):
    n, d = idx.shape[0], table.shape[1]
    info = plsc.get_sparse_core_info()
    nw = info.num_cores * info.num_subcores
    npw = n // nw
    assert n % nw == 0 and npw % csize == 0 and csize % 8 == 0 and csize <= 128
    nchunk = npw // csize
    nbuf = min(_SC_NBUF, nchunk)
    mesh = plsc.VectorSubcoreMesh(core_axis_name="c", subcore_axis_name="s")

    @functools.partial(
        pl.kernel, mesh=mesh,
        out_type=jax.ShapeDtypeStruct((n, d), table.dtype),
        scratch_types=(
            [pltpu.VMEM((npw,), jnp.int32)]
            + [pltpu.VMEM((csize, d), table.dtype) for _ in range(nbuf)]
            + [pltpu.SemaphoreType.DMA for _ in range(2 * nbuf)]
        ),
    )
    def k(table_hbm, idx_hbm, out_hbm, idx_v, *bufs_sems):
        bufs = bufs_sems[:nbuf]
        gsem = bufs_sems[nbuf:2 * nbuf]
        wsem = bufs_sems[2 * nbuf:]
        wid = lax.axis_index("s") * info.num_cores + lax.axis_index("c")
        base = wid * npw
        pltpu.sync_copy(idx_hbm.at[pl.ds(base, npw)], idx_v)
        gh, wh = {}, {}
        for c in range(nchunk):
            bi = c % nbuf
            if c >= nbuf:
                wh[c - nbuf].wait()
            gh[c] = pltpu.make_async_copy(
                table_hbm.at[idx_v.at[pl.ds(c * csize, csize)]],
                bufs[bi], gsem[bi])
            gh[c].start()
            if c >= 1:
                gh[c - 1].wait()
                wh[c - 1] = pltpu.make_async_copy(
                    bufs[(c - 1) % nbuf],
                    out_hbm.at[pl.ds(base + (c - 1) * csize, csize)],
                    wsem[(c - 1) % nbuf])
                wh[c - 1].start()
        gh[nchunk - 1].wait()
        wh[nchunk - 1] = pltpu.make_async_copy(
            bufs[(nchunk - 1) % nbuf],
            out_hbm.at[pl.ds(base + (nchunk - 1) * csize, csize)],
            wsem[(nchunk - 1) % nbuf])
        wh[nchunk - 1].start()
        for c in range(max(0, nchunk - nbuf), nchunk):
            wh[c].wait()

    return k(table, idx)


def _embed_kernel(tok_ref, emb_ref, out_ref):
    tok = tok_ref[0]
    vocab = emb_ref.shape[0]
    iota = jax.lax.broadcasted_iota(jnp.int32, (vocab, 1), 0)
    onehot_t = (iota == tok).astype(jnp.float32)
    out_ref[0] = jax.lax.dot_general(
        onehot_t, emb_ref[...], (((0,), (0,)), ((), ())),
        preferred_element_type=jnp.float32)


def _recurrence_kernel(e0_ref, e1_ref, e2_ref, wx_ref, wl_ref, wu_ref,
                       b_ref, lns_ref, lnb_ref, hinit_ref, hout_ref, H, C):
    t = pl.program_id(0)
    h = HID_

    @pl.when(t == 0)
    def _init():
        H[...] = jnp.broadcast_to(hinit_ref[...], (M_ + 1, B_, h))
        C[...] = jnp.zeros((M_ + 1, B_, h), jnp.float32)

    e0 = e0_ref[0]
    e1 = e1_ref[0]
    e2 = e2_ref[0]
    x1 = e1[1:].reshape(ROWS_, h)
    x2 = e0[:M_].reshape(ROWS_, h)
    x3 = e1[:M_].reshape(ROWS_, h)
    x4 = e2[:M_].reshape(ROWS_, h)
    h_left = H[1:].reshape(ROWS_, h)
    h_up = H[:M_].reshape(ROWS_, h)
    c_left = C[1:].reshape(ROWS_, h)
    c_up = C[:M_].reshape(ROWS_, h)

    f32 = jnp.float32
    g = (jnp.dot(x1, wx_ref[0:h], preferred_element_type=f32)
         + jnp.dot(x2, wx_ref[h:2 * h], preferred_element_type=f32)
         + jnp.dot(x3, wx_ref[2 * h:3 * h], preferred_element_type=f32)
         + jnp.dot(x4, wx_ref[3 * h:4 * h], preferred_element_type=f32)
         + jnp.dot(h_left, wl_ref[...], preferred_element_type=f32)
         + jnp.dot(h_up, wu_ref[...], preferred_element_type=f32)
         + b_ref[...])
    fc = jax.nn.sigmoid(g[:, 0:h])
    fr = jax.nn.sigmoid(g[:, h:2 * h])
    ig = jax.nn.sigmoid(g[:, 2 * h:3 * h])
    og = jax.nn.sigmoid(g[:, 3 * h:4 * h])
    cg = jnp.tanh(g[:, 4 * h:5 * h])

    c_new = fc * c_left + fr * c_up + ig * cg
    mu = jnp.mean(c_new, axis=-1, keepdims=True)
    var = jnp.mean((c_new - mu) * (c_new - mu), axis=-1, keepdims=True)
    zn = (c_new - mu) * jax.lax.rsqrt(var + 1e-6) * lns_ref[...] + lnb_ref[...]
    s = og * jnp.tanh(zn)

    s3 = s.reshape(M_, B_, h)
    c3 = c_new.reshape(M_, B_, h)
    ii = jax.lax.broadcasted_iota(jnp.int32, (M_, 1, 1), 0)
    mask = (ii <= t) & (ii >= t - (N_ - 1))
    H[1:] = jnp.where(mask, s3, H[1:])
    C[1:] = jnp.where(mask, c3, C[1:])
    hout_ref[0] = s3


def _head_kernel(h_ref, w_ref, b_ref, o_ref):
    o_ref[...] = (jnp.dot(h_ref[...], w_ref[...],
                          preferred_element_type=jnp.float32) + b_ref[...])


def _tok_index():
    q = np.arange(Q_)[:, None, None]
    a = np.arange(M_ + 1)[None, :, None]
    bb = np.arange(B_)[None, None, :]
    c = np.clip(q - a, 0, N_ + 1)
    return ((a * (N_ + 2) + c) * B_ + bb).reshape(-1).astype(np.int32)


def _undiag_index():
    ii = np.arange(M_)[:, None, None]
    jj = np.arange(N_)[None, :, None]
    bb = np.arange(B_)[None, None, :]
    return (((ii + jj) * M_ + ii) * B_ + bb).reshape(-1).astype(np.int32)


def kernel(x_bmn, embed, gate_w, gate_b, ln_scale, ln_bias, h_init,
           head_w, head_b):
    b, m, n = x_bmn.shape
    h = h_init.shape[1]

    x_p = jnp.pad(x_bmn, ((0, 0), (1, 0), (1, 1)))
    xp_flat = jnp.transpose(x_p, (1, 2, 0)).reshape(-1)
    tok = xp_flat[jnp.asarray(_tok_index())]

    ntok = Q_ * (m + 1) * b
    eblk = 2048
    ed_flat = pl.pallas_call(
        _embed_kernel,
        grid=(ntok // eblk,),
        in_specs=[
            pl.BlockSpec((1, 1, eblk), lambda i: (i, 0, 0)),
            pl.BlockSpec((2 * h, h), lambda i: (0, 0)),
        ],
        out_specs=pl.BlockSpec((1, eblk, h), lambda i: (i, 0, 0)),
        out_shape=jax.ShapeDtypeStruct((ntok // eblk, eblk, h), jnp.float32),
    )(tok.reshape(ntok // eblk, 1, eblk), embed)
    Ed = ed_flat.reshape(Q_, m + 1, b, h)

    wx = gate_w[:4 * h]
    wl = gate_w[4 * h:5 * h]
    wu = gate_w[5 * h:]
    b2 = gate_b.reshape(1, 5 * h)
    lns2 = ln_scale.reshape(1, h)
    lnb2 = ln_bias.reshape(1, h)

    ed_spec = lambda off: pl.BlockSpec(
        (1, m + 1, B_, h), lambda t, o=off: (t + o, 0, 0, 0))
    h_diag = pl.pallas_call(
        _recurrence_kernel,
        grid=(T_,),
        in_specs=[
            ed_spec(0), ed_spec(1), ed_spec(2),
            pl.BlockSpec((4 * h, 5 * h), lambda t: (0, 0)),
            pl.BlockSpec((h, 5 * h), lambda t: (0, 0)),
            pl.BlockSpec((h, 5 * h), lambda t: (0, 0)),
            pl.BlockSpec((1, 5 * h), lambda t: (0, 0)),
            pl.BlockSpec((1, h), lambda t: (0, 0)),
            pl.BlockSpec((1, h), lambda t: (0, 0)),
            pl.BlockSpec((1, h), lambda t: (0, 0)),
        ],
        out_specs=pl.BlockSpec((1, M_, B_, h), lambda t: (t, 0, 0, 0)),
        out_shape=jax.ShapeDtypeStruct((T_, M_, B_, h), jnp.float32),
        scratch_shapes=[pltpu.VMEM((M_ + 1, B_, h), jnp.float32),
                        pltpu.VMEM((M_ + 1, B_, h), jnp.float32)],
        compiler_params=pltpu.CompilerParams(
            dimension_semantics=("arbitrary",)),
    )(Ed, Ed, Ed, wx, wl, wu, b2, lns2, lnb2, h_init)

    i2, j2 = np.meshgrid(np.arange(m), np.arange(n), indexing='ij')
    h_rows = h_diag[i2 + j2, i2].reshape(b * m * n, h)

    nblk = 8
    blk = (b * m * n) // nblk
    logits = pl.pallas_call(
        _head_kernel,
        grid=(nblk,),
        in_specs=[
            pl.BlockSpec((blk, h), lambda i: (i, 0)),
            pl.BlockSpec((h, 2 * h), lambda i: (0, 0)),
            pl.BlockSpec((1, 2 * h), lambda i: (0, 0)),
        ],
        out_specs=pl.BlockSpec((blk, 2 * h), lambda i: (i, 0)),
        out_shape=jax.ShapeDtypeStruct((b * m * n, 2 * h), jnp.float32),
    )(h_rows, head_w, head_b.reshape(1, 2 * h))

    return logits.reshape(b, m, n, 2 * h)


# speedup vs baseline: 2.6360x; 2.6360x over previous
import functools
import numpy as np
import jax
from jax import lax
import jax.numpy as jnp
from jax.experimental import pallas as pl
from jax.experimental.pallas import tpu as pltpu
from jax.experimental.pallas import tpu_sc as plsc

HID_ = 128
M_, N_, B_ = 28, 28, 32
T_ = M_ + N_ - 1
Q_ = 64
ROWS_ = M_ * B_

_SC_NBUF = 3


def _sc_gather(table, idx, csize):
    n, d = idx.shape[0], table.shape[1]
    info = plsc.get_sparse_core_info()
    nw = info.num_cores * info.num_subcores
    npw = n // nw
    assert n % nw == 0 and npw % csize == 0 and csize % 8 == 0 and csize <= 128
    nchunk = npw // csize
    nbuf = min(_SC_NBUF, nchunk)
    mesh = plsc.VectorSubcoreMesh(core_axis_name="c", subcore_axis_name="s")

    @functools.partial(
        pl.kernel, mesh=mesh,
        out_type=jax.ShapeDtypeStruct((n, d), table.dtype),
        scratch_types=(
            [pltpu.VMEM((npw,), jnp.int32)]
            + [pltpu.VMEM((csize, d), table.dtype) for _ in range(nbuf)]
            + [pltpu.SemaphoreType.DMA for _ in range(2 * nbuf)]
        ),
    )
    def k(table_hbm, idx_hbm, out_hbm, idx_v, *bufs_sems):
        bufs = bufs_sems[:nbuf]
        gsem = bufs_sems[nbuf:2 * nbuf]
        wsem = bufs_sems[2 * nbuf:]
        wid = lax.axis_index("s") * info.num_cores + lax.axis_index("c")
        base = wid * npw
        pltpu.sync_copy(idx_hbm.at[pl.ds(base, npw)], idx_v)
        gh, wh = {}, {}
        for c in range(nchunk):
            bi = c % nbuf
            if c >= nbuf:
                wh[c - nbuf].wait()
            gh[c] = pltpu.make_async_copy(
                table_hbm.at[idx_v.at[pl.ds(c * csize, csize)]],
                bufs[bi], gsem[bi])
            gh[c].start()
            if c >= 1:
                gh[c - 1].wait()
                wh[c - 1] = pltpu.make_async_copy(
                    bufs[(c - 1) % nbuf],
                    out_hbm.at[pl.ds(base + (c - 1) * csize, csize)],
                    wsem[(c - 1) % nbuf])
                wh[c - 1].start()
        gh[nchunk - 1].wait()
        wh[nchunk - 1] = pltpu.make_async_copy(
            bufs[(nchunk - 1) % nbuf],
            out_hbm.at[pl.ds(base + (nchunk - 1) * csize, csize)],
            wsem[(nchunk - 1) % nbuf])
        wh[nchunk - 1].start()
        for c in range(max(0, nchunk - nbuf), nchunk):
            wh[c].wait()

    return k(table, idx)


HID_ = 128
M_, N_, B_ = 28, 28, 32
T_ = M_ + N_ - 1
Q_ = T_ + 2
ROWS_ = M_ * B_


def _recurrence_kernel(e0_ref, e1_ref, e2_ref, wx_ref, wl_ref, wu_ref,
                       b_ref, lns_ref, lnb_ref, hinit_ref, hout_ref, H, C):
    t = pl.program_id(0)
    h = HID_

    @pl.when(t == 0)
    def _init():
        H[...] = jnp.broadcast_to(hinit_ref[...], (M_ + 1, B_, h))
        C[...] = jnp.zeros((M_ + 1, B_, h), jnp.float32)

    e0 = e0_ref[0]
    e1 = e1_ref[0]
    e2 = e2_ref[0]
    x1 = e1[1:].reshape(ROWS_, h)
    x2 = e0[:M_].reshape(ROWS_, h)
    x3 = e1[:M_].reshape(ROWS_, h)
    x4 = e2[:M_].reshape(ROWS_, h)
    h_left = H[1:].reshape(ROWS_, h)
    h_up = H[:M_].reshape(ROWS_, h)
    c_left = C[1:].reshape(ROWS_, h)
    c_up = C[:M_].reshape(ROWS_, h)

    f32 = jnp.float32
    g = (jnp.dot(x1, wx_ref[0:h], preferred_element_type=f32)
         + jnp.dot(x2, wx_ref[h:2 * h], preferred_element_type=f32)
         + jnp.dot(x3, wx_ref[2 * h:3 * h], preferred_element_type=f32)
         + jnp.dot(x4, wx_ref[3 * h:4 * h], preferred_element_type=f32)
         + jnp.dot(h_left, wl_ref[...], preferred_element_type=f32)
         + jnp.dot(h_up, wu_ref[...], preferred_element_type=f32)
         + b_ref[...])
    fc = jax.nn.sigmoid(g[:, 0:h])
    fr = jax.nn.sigmoid(g[:, h:2 * h])
    ig = jax.nn.sigmoid(g[:, 2 * h:3 * h])
    og = jax.nn.sigmoid(g[:, 3 * h:4 * h])
    cg = jnp.tanh(g[:, 4 * h:5 * h])

    c_new = fc * c_left + fr * c_up + ig * cg
    mu = jnp.mean(c_new, axis=-1, keepdims=True)
    var = jnp.mean((c_new - mu) * (c_new - mu), axis=-1, keepdims=True)
    zn = (c_new - mu) * jax.lax.rsqrt(var + 1e-6) * lns_ref[...] + lnb_ref[...]
    s = og * jnp.tanh(zn)

    s3 = s.reshape(M_, B_, h)
    c3 = c_new.reshape(M_, B_, h)
    ii = jax.lax.broadcasted_iota(jnp.int32, (M_, 1, 1), 0)
    mask = (ii <= t) & (ii >= t - (N_ - 1))
    H[1:] = jnp.where(mask, s3, H[1:])
    C[1:] = jnp.where(mask, c3, C[1:])
    hout_ref[0] = s3


def _head_kernel(h_ref, w_ref, b_ref, o_ref):
    o_ref[...] = (jnp.dot(h_ref[...], w_ref[...],
                          preferred_element_type=jnp.float32) + b_ref[...])


def _undiag_index():
    ii = np.arange(M_)[:, None, None]
    jj = np.arange(N_)[None, :, None]
    bb = np.arange(B_)[None, None, :]
    return (((ii + jj) * M_ + ii) * B_ + bb).reshape(-1).astype(np.int32)


def kernel(x_bmn, embed, gate_w, gate_b, ln_scale, ln_bias, h_init,
           head_w, head_b):
    b, m, n = x_bmn.shape
    h = h_init.shape[1]

    x_p = jnp.pad(x_bmn, ((0, 0), (1, 0), (1, 1)))
    q_i = np.arange(Q_)[:, None]
    a_i = np.arange(m + 1)[None, :]
    c_i = np.clip(q_i - a_i, 0, n + 1)
    tok = jnp.transpose(x_p[:, a_i, c_i], (1, 2, 0))
    Ed = jnp.take(embed.astype(jnp.bfloat16), tok, axis=0)

    wx = gate_w[:4 * h].astype(jnp.bfloat16)
    wl = gate_w[4 * h:5 * h]
    wu = gate_w[5 * h:]
    b2 = gate_b.reshape(1, 5 * h)
    lns2 = ln_scale.reshape(1, h)
    lnb2 = ln_bias.reshape(1, h)

    ed_spec = lambda off: pl.BlockSpec(
        (1, m + 1, B_, h), lambda t, o=off: (t + o, 0, 0, 0))
    h_diag = pl.pallas_call(
        _recurrence_kernel,
        grid=(T_,),
        in_specs=[
            ed_spec(0), ed_spec(1), ed_spec(2),
            pl.BlockSpec((4 * h, 5 * h), lambda t: (0, 0)),
            pl.BlockSpec((h, 5 * h), lambda t: (0, 0)),
            pl.BlockSpec((h, 5 * h), lambda t: (0, 0)),
            pl.BlockSpec((1, 5 * h), lambda t: (0, 0)),
            pl.BlockSpec((1, h), lambda t: (0, 0)),
            pl.BlockSpec((1, h), lambda t: (0, 0)),
            pl.BlockSpec((1, h), lambda t: (0, 0)),
        ],
        out_specs=pl.BlockSpec((1, M_, B_, h), lambda t: (t, 0, 0, 0)),
        out_shape=jax.ShapeDtypeStruct((T_, M_, B_, h), jnp.float32),
        scratch_shapes=[pltpu.VMEM((M_ + 1, B_, h), jnp.float32),
                        pltpu.VMEM((M_ + 1, B_, h), jnp.float32)],
        compiler_params=pltpu.CompilerParams(
            dimension_semantics=("arbitrary",)),
    )(Ed, Ed, Ed, wx, wl, wu, b2, lns2, lnb2, h_init)

    h_rows = _sc_gather(h_diag.reshape(T_ * M_ * B_, h),
                        jnp.asarray(_undiag_index()), csize=112)

    nblk = 8
    blk = (b * m * n) // nblk
    logits = pl.pallas_call(
        _head_kernel,
        grid=(nblk,),
        in_specs=[
            pl.BlockSpec((blk, h), lambda i: (i, 0)),
            pl.BlockSpec((h, 2 * h), lambda i: (0, 0)),
            pl.BlockSpec((1, 2 * h), lambda i: (0, 0)),
        ],
        out_specs=pl.BlockSpec((blk, 2 * h), lambda i: (i, 0)),
        out_shape=jax.ShapeDtypeStruct((b * m * n, 2 * h), jnp.float32),
    )(h_rows, head_w, head_b.reshape(1, 2 * h))

    return logits.reshape(b, m, n, 2 * h)


# speedup vs baseline: 2.6385x; 1.0010x over previous
import functools
import numpy as np
import jax
from jax import lax
import jax.numpy as jnp
from jax.experimental import pallas as pl
from jax.experimental.pallas import tpu as pltpu
from jax.experimental.pallas import tpu_sc as plsc

HID_ = 128
M_, N_, B_ = 28, 28, 32
T_ = M_ + N_ - 1
Q_ = T_ + 2
ROWS_ = M_ * B_

_SC_NBUF = 3


def _sc_gather(table, idx, csize):
    n, d = idx.shape[0], table.shape[1]
    info = plsc.get_sparse_core_info()
    nw = info.num_cores * info.num_subcores
    npw = n // nw
    assert n % nw == 0 and npw % csize == 0 and csize % 8 == 0 and csize <= 128
    nchunk = npw // csize
    nbuf = min(_SC_NBUF, nchunk)
    mesh = plsc.VectorSubcoreMesh(core_axis_name="c", subcore_axis_name="s")

    @functools.partial(
        pl.kernel, mesh=mesh,
        out_type=jax.ShapeDtypeStruct((n, d), table.dtype),
        scratch_types=(
            [pltpu.VMEM((npw,), jnp.int32)]
            + [pltpu.VMEM((csize, d), table.dtype) for _ in range(nbuf)]
            + [pltpu.SemaphoreType.DMA for _ in range(2 * nbuf)]
        ),
    )
    def k(table_hbm, idx_hbm, out_hbm, idx_v, *bufs_sems):
        bufs = bufs_sems[:nbuf]
        gsem = bufs_sems[nbuf:2 * nbuf]
        wsem = bufs_sems[2 * nbuf:]
        wid = lax.axis_index("s") * info.num_cores + lax.axis_index("c")
        base = wid * npw
        pltpu.sync_copy(idx_hbm.at[pl.ds(base, npw)], idx_v)
        gh, wh = {}, {}
        for c in range(nchunk):
            bi = c % nbuf
            if c >= nbuf:
                wh[c - nbuf].wait()
            gh[c] = pltpu.make_async_copy(
                table_hbm.at[idx_v.at[pl.ds(c * csize, csize)]],
                bufs[bi], gsem[bi])
            gh[c].start()
            if c >= 1:
                gh[c - 1].wait()
                wh[c - 1] = pltpu.make_async_copy(
                    bufs[(c - 1) % nbuf],
                    out_hbm.at[pl.ds(base + (c - 1) * csize, csize)],
                    wsem[(c - 1) % nbuf])
                wh[c - 1].start()
        gh[nchunk - 1].wait()
        wh[nchunk - 1] = pltpu.make_async_copy(
            bufs[(nchunk - 1) % nbuf],
            out_hbm.at[pl.ds(base + (nchunk - 1) * csize, csize)],
            wsem[(nchunk - 1) % nbuf])
        wh[nchunk - 1].start()
        for c in range(max(0, nchunk - nbuf), nchunk):
            wh[c].wait()

    return k(table, idx)


def _recurrence_kernel(e0_ref, e1_ref, e2_ref, wx_ref, wl_ref, wu_ref,
                       b_ref, lns_ref, lnb_ref, hinit_ref, hout_ref, H, C):
    t = pl.program_id(0)
    h = HID_

    @pl.when(t == 0)
    def _init():
        H[...] = jnp.broadcast_to(hinit_ref[...], (M_ + 1, B_, h))
        C[...] = jnp.zeros((M_ + 1, B_, h), jnp.float32)

    e0 = e0_ref[0]
    e1 = e1_ref[0]
    e2 = e2_ref[0]
    x1 = e1[1:].reshape(ROWS_, h)
    x2 = e0[:M_].reshape(ROWS_, h)
    x3 = e1[:M_].reshape(ROWS_, h)
    x4 = e2[:M_].reshape(ROWS_, h)
    h_left = H[1:].reshape(ROWS_, h)
    h_up = H[:M_].reshape(ROWS_, h)
    c_left = C[1:].reshape(ROWS_, h)
    c_up = C[:M_].reshape(ROWS_, h)

    f32 = jnp.float32
    g = (jnp.dot(x1, wx_ref[0:h], preferred_element_type=f32)
         + jnp.dot(x2, wx_ref[h:2 * h], preferred_element_type=f32)
         + jnp.dot(x3, wx_ref[2 * h:3 * h], preferred_element_type=f32)
         + jnp.dot(x4, wx_ref[3 * h:4 * h], preferred_element_type=f32)
         + jnp.dot(h_left, wl_ref[...], preferred_element_type=f32)
         + jnp.dot(h_up, wu_ref[...], preferred_element_type=f32)
         + b_ref[...])
    fc = jax.nn.sigmoid(g[:, 0:h])
    fr = jax.nn.sigmoid(g[:, h:2 * h])
    ig = jax.nn.sigmoid(g[:, 2 * h:3 * h])
    og = jax.nn.sigmoid(g[:, 3 * h:4 * h])
    cg = jnp.tanh(g[:, 4 * h:5 * h])

    c_new = fc * c_left + fr * c_up + ig * cg
    mu = jnp.mean(c_new, axis=-1, keepdims=True)
    var = jnp.mean((c_new - mu) * (c_new - mu), axis=-1, keepdims=True)
    zn = (c_new - mu) * jax.lax.rsqrt(var + 1e-6) * lns_ref[...] + lnb_ref[...]
    s = og * jnp.tanh(zn)

    s3 = s.reshape(M_, B_, h)
    c3 = c_new.reshape(M_, B_, h)
    ii = jax.lax.broadcasted_iota(jnp.int32, (M_, 1, 1), 0)
    mask = (ii <= t) & (ii >= t - (N_ - 1))
    H[1:] = jnp.where(mask, s3, H[1:])
    C[1:] = jnp.where(mask, c3, C[1:])
    hout_ref[0] = s3


def _head_kernel(h_ref, w_ref, b_ref, o_ref):
    o_ref[...] = (jnp.dot(h_ref[...], w_ref[...],
                          preferred_element_type=jnp.float32) + b_ref[...])


def _undiag_index():
    ii = np.arange(M_)[:, None, None]
    jj = np.arange(N_)[None, :, None]
    bb = np.arange(B_)[None, None, :]
    return (((ii + jj) * M_ + ii) * B_ + bb).reshape(-1).astype(np.int32)


def kernel(x_bmn, embed, gate_w, gate_b, ln_scale, ln_bias, h_init,
           head_w, head_b):
    b, m, n = x_bmn.shape
    h = h_init.shape[1]

    x_p = jnp.pad(x_bmn, ((0, 0), (1, 0), (1, 1)))
    q_i = np.arange(Q_)[:, None]
    a_i = np.arange(m + 1)[None, :]
    c_i = np.clip(q_i - a_i, 0, n + 1)
    tok = jnp.transpose(x_p[:, a_i, c_i], (1, 2, 0))
    Ed = jnp.take(embed.astype(jnp.bfloat16), tok, axis=0)

    wx = gate_w[:4 * h].astype(jnp.bfloat16)
    wl = gate_w[4 * h:5 * h]
    wu = gate_w[5 * h:]
    b2 = gate_b.reshape(1, 5 * h)
    lns2 = ln_scale.reshape(1, h)
    lnb2 = ln_bias.reshape(1, h)

    ed_spec = lambda off: pl.BlockSpec(
        (1, m + 1, B_, h), lambda t, o=off: (t + o, 0, 0, 0))
    h_diag = pl.pallas_call(
        _recurrence_kernel,
        grid=(T_,),
        in_specs=[
            ed_spec(0), ed_spec(1), ed_spec(2),
            pl.BlockSpec((4 * h, 5 * h), lambda t: (0, 0)),
            pl.BlockSpec((h, 5 * h), lambda t: (0, 0)),
            pl.BlockSpec((h, 5 * h), lambda t: (0, 0)),
            pl.BlockSpec((1, 5 * h), lambda t: (0, 0)),
            pl.BlockSpec((1, h), lambda t: (0, 0)),
            pl.BlockSpec((1, h), lambda t: (0, 0)),
            pl.BlockSpec((1, h), lambda t: (0, 0)),
        ],
        out_specs=pl.BlockSpec((1, M_, B_, h), lambda t: (t, 0, 0, 0)),
        out_shape=jax.ShapeDtypeStruct((T_, M_, B_, h), jnp.float32),
        scratch_shapes=[pltpu.VMEM((M_ + 1, B_, h), jnp.float32),
                        pltpu.VMEM((M_ + 1, B_, h), jnp.float32)],
        compiler_params=pltpu.CompilerParams(
            dimension_semantics=("arbitrary",)),
    )(Ed, Ed, Ed, wx, wl, wu, b2, lns2, lnb2, h_init)

    h_rows = _sc_gather(h_diag.reshape(T_ * M_ * B_, h),
                        jnp.asarray(_undiag_index()), csize=112)

    nblk = 8
    blk = (b * m * n) // nblk
    logits = pl.pallas_call(
        _head_kernel,
        grid=(nblk,),
        in_specs=[
            pl.BlockSpec((blk, h), lambda i: (i, 0)),
            pl.BlockSpec((h, 2 * h), lambda i: (0, 0)),
            pl.BlockSpec((1, 2 * h), lambda i: (0, 0)),
        ],
        out_specs=pl.BlockSpec((blk, 2 * h), lambda i: (i, 0)),
        out_shape=jax.ShapeDtypeStruct((b * m * n, 2 * h), jnp.float32),
    )(h_rows, head_w, head_b.reshape(1, 2 * h))

    return logits.reshape(b, m, n, 2 * h)
